# Initial kernel scaffold; baseline (speedup 1.0000x reference)
import jax
import jax.numpy as jnp
from jax.experimental import pallas as pl


def kernel(x, train_pos_edge_index, edge_index, edge_weight, pos_edge_index, neg_edge_index, W1, b1, W2, b2, W_attr, b_attr, W_att, b_att):
    raise NotImplementedError("write your pallas kernel here")


# speedup vs baseline: 16.0153x; 16.0153x over previous
import functools

import jax
import jax.numpy as jnp
from jax import lax
from jax.experimental import pallas as pl
from jax.experimental.pallas import tpu as pltpu
from jax.experimental.pallas import tpu_sc as plsc

N_NODES = 10000
N_PAD = 10112
NCORE = 2
NSUB = 16
NW = NCORE * NSUB
STRIPE = N_PAD // NSUB
ECH = 128
E_TOT = 320000
CH_E = 79
EPAD_E = NW * CH_E * ECH
EL_TOT = 200000
CH_L = 49
EPAD_L = NW * CH_L * ECH

F32 = jnp.float32
I32 = jnp.int32

_MESH = plsc.VectorSubcoreMesh(core_axis_name="c", subcore_axis_name="s")
_SC_PARAMS = pltpu.CompilerParams(needs_layout_passes=False,
                                  use_tc_tiling_on_sc=False)


def _sds(shape, dtype):
    return jax.ShapeDtypeStruct(shape, dtype)


def _zero_buf(buf, nrow, ncol16):
    z = jnp.zeros((16,), F32)

    def body(r, c):
        for k in range(ncol16):
            buf[r, pl.ds(k * 16, 16)] = z
        return c

    lax.fori_loop(0, nrow, body, 0)


def _zero_stripe(rows, acc, sid):
    base = pl.multiple_of(sid * STRIPE, 8)
    for k in range(4):
        pltpu.sync_copy(rows, acc.at[pl.ds(base + k * 128, 128)])
    rem = STRIPE - 512
    pltpu.sync_copy(rows.at[pl.ds(0, rem)], acc.at[pl.ds(base + 512, rem)])


def _dump_stripe(acc, part, cid, sid):
    base = pl.multiple_of(sid * STRIPE, 8)
    for k in range(4):
        pltpu.sync_copy(acc.at[pl.ds(base + k * 128, 128)],
                        part.at[cid].at[pl.ds(base + k * 128, 128)])
    rem = STRIPE - 512
    pltpu.sync_copy(acc.at[pl.ds(base + 512, rem)],
                    part.at[cid].at[pl.ds(base + 512, rem)])


def _deg_body(d1_h, d2_h, w_h, deg1_h, deg2_h, buf1, buf2, bufw, ldeg1, ldeg2):
    cid = lax.axis_index("c")
    sid = lax.axis_index("s")
    wid = cid * NSUB + sid
    pltpu.sync_copy(d1_h.at[wid], buf1)
    pltpu.sync_copy(d2_h.at[wid], buf2)
    pltpu.sync_copy(w_h.at[wid], bufw)
    z = jnp.zeros((16,), F32)

    def zb(i, c):
        ldeg1[pl.ds(i * 16, 16)] = z
        ldeg2[pl.ds(i * 16, 16)] = z
        return c

    lax.fori_loop(0, N_PAD // 16, zb, 0)
    ones = jnp.ones((16,), F32)

    def eb(j, c):
        for k in range(8):
            i1 = buf1[j, pl.ds(k * 16, 16)]
            plsc.addupdate_scatter(ldeg1, [i1], ones)
            i2 = buf2[j, pl.ds(k * 16, 16)]
            wv = bufw[j, pl.ds(k * 16, 16)]
            plsc.addupdate_scatter(ldeg2, [i2], wv)
        return c

    lax.fori_loop(0, CH_E, eb, 0)
    pltpu.sync_copy(ldeg1, deg1_h.at[wid])
    pltpu.sync_copy(ldeg2, deg2_h.at[wid])


def _deg_call(dst1, dst2, wpad):
    return pl.kernel(
        _deg_body,
        out_type=(_sds((NW, N_PAD), F32), _sds((NW, N_PAD), F32)),
        mesh=_MESH,
        compiler_params=_SC_PARAMS,
        scratch_types=[
            pltpu.VMEM((CH_E, ECH), I32),
            pltpu.VMEM((CH_E, ECH), I32),
            pltpu.VMEM((CH_E, ECH), F32),
            pltpu.VMEM((N_PAD,), F32),
            pltpu.VMEM((N_PAD,), F32),
        ],
    )(dst1, dst2, wpad)


def _make_prop_body(F):
    ncol16 = F // 16

    def body(src_h, dst_h, y_h, part_h, sidx, didx, rows, acc, sem):
        cid = lax.axis_index("c")
        sid = lax.axis_index("s")
        wid = cid * NSUB + sid
        pltpu.sync_copy(src_h.at[wid], sidx)
        pltpu.sync_copy(dst_h.at[wid], didx)
        _zero_buf(rows, ECH, ncol16)
        _zero_stripe(rows, acc, sid)
        plsc.subcore_barrier()

        def chunk(j, c):
            pltpu.async_copy(y_h.at[sidx.at[j]], rows, sem).wait()
            pltpu.sync_copy(rows, acc.at[didx.at[j]], add=True)
            return c

        lax.fori_loop(0, CH_E, chunk, 0)
        plsc.subcore_barrier()
        _dump_stripe(acc, part_h, cid, sid)

    return body


def _prop_call(F, src, dst, y):
    return pl.kernel(
        _make_prop_body(F),
        out_type=_sds((NCORE, N_PAD, F), F32),
        mesh=_MESH,
        compiler_params=_SC_PARAMS,
        scratch_types=[
            pltpu.VMEM((CH_E, ECH), I32),
            pltpu.VMEM((CH_E, ECH), I32),
            pltpu.VMEM((ECH, F), F32),
            pltpu.VMEM_SHARED((N_PAD, F), F32),
            pltpu.SemaphoreType.DMA,
        ],
    )(src, dst, y)


def _prop3dot_body(src_h, dst_h, w_h, y_h, si_h, di_h, h2_h, part_h, res_h,
                   sidx, didx, wbuf, rows, si, di, rs, rd, rbuf, acc, sem):
    cid = lax.axis_index("c")
    sid = lax.axis_index("s")
    wid = cid * NSUB + sid
    pltpu.sync_copy(src_h.at[wid], sidx)
    pltpu.sync_copy(dst_h.at[wid], didx)
    pltpu.sync_copy(w_h.at[wid], wbuf)
    pltpu.sync_copy(si_h.at[wid], si)
    pltpu.sync_copy(di_h.at[wid], di)
    _zero_buf(rows, ECH, 2)
    _zero_stripe(rows, acc, sid)
    plsc.subcore_barrier()

    def chunk(j, c):
        pltpu.async_copy(y_h.at[sidx.at[j]], rows, sem).wait()

        def sgroup(g, c2):
            wv = wbuf[j, pl.ds(g * 16, 16)]
            for l in range(16):
                w = wv[l]
                r = g * 16 + l
                for k in range(2):
                    rows[r, pl.ds(k * 16, 16)] = rows[r, pl.ds(k * 16, 16)] * w
            return c2

        lax.fori_loop(0, ECH // 16, sgroup, 0)
        pltpu.sync_copy(rows, acc.at[didx.at[j]], add=True)
        return c

    lax.fori_loop(0, CH_E, chunk, 0)

    lane = lax.iota(I32, 16)

    def dchunk(j, c):
        pltpu.async_copy(h2_h.at[si.at[j]], rs, sem).wait()
        pltpu.async_copy(h2_h.at[di.at[j]], rd, sem).wait()

        def group(g, c2):
            res = jnp.zeros((16,), F32)
            for l in range(16):
                e = g * 16 + l
                av = rs[e, pl.ds(0, 16)] * rd[e, pl.ds(0, 16)]
                for k in range(1, 4):
                    av = av + rs[e, pl.ds(k * 16, 16)] * rd[e, pl.ds(k * 16, 16)]
                s = jnp.sum(av)
                res = jnp.where(lane == l, s, res)
            rbuf[j, pl.ds(g * 16, 16)] = res
            return c2

        lax.fori_loop(0, ECH // 16, group, 0)
        return c

    lax.fori_loop(0, CH_L, dchunk, 0)
    pltpu.sync_copy(rbuf, res_h.at[wid])

    plsc.subcore_barrier()
    _dump_stripe(acc, part_h, cid, sid)


def _prop3dot_call(src, dst, wpad, y, si, di, h2):
    return pl.kernel(
        _prop3dot_body,
        out_type=(_sds((NCORE, N_PAD, 32), F32), _sds((NW, CH_L, ECH), F32)),
        mesh=_MESH,
        compiler_params=_SC_PARAMS,
        scratch_types=[
            pltpu.VMEM((CH_E, ECH), I32),
            pltpu.VMEM((CH_E, ECH), I32),
            pltpu.VMEM((CH_E, ECH), F32),
            pltpu.VMEM((ECH, 32), F32),
            pltpu.VMEM((CH_L, ECH), I32),
            pltpu.VMEM((CH_L, ECH), I32),
            pltpu.VMEM((ECH, 64), F32),
            pltpu.VMEM((ECH, 64), F32),
            pltpu.VMEM((CH_L, ECH), F32),
            pltpu.VMEM_SHARED((N_PAD, 32), F32),
            pltpu.SemaphoreType.DMA,
        ],
    )(src, dst, wpad, y, si, di, h2)


_HIGH = lax.Precision.HIGHEST


def _mm(a, b):
    return jnp.dot(a, b, precision=_HIGH, preferred_element_type=F32)


def _dinv_body(d1_ref, d2_ref, o1_ref, o2_ref):
    o1_ref[...] = lax.rsqrt(jnp.sum(d1_ref[...], axis=0, keepdims=True) + 1.0)
    o2_ref[...] = lax.rsqrt(jnp.sum(d2_ref[...], axis=0, keepdims=True) + 1.0)


def _lin1_body(x_ref, w1_ref, dinv_ref, o_ref):
    o_ref[...] = dinv_ref[...] * _mm(x_ref[...], w1_ref[...])


def _lin2_body(s1a, s1b, xw1p, dinv, b1, w2, o_ref):
    h1 = jax.nn.relu(dinv[...] * (s1a[...] + s1b[...] + xw1p[...]) + b1[...])
    o_ref[...] = dinv[...] * _mm(h1, w2[...])


def _lin3_body(s2a, s2b, y2p, dinv1, b2, wcat, dinv2, h2_o, y3p_o):
    h2 = dinv1[...] * (s2a[...] + s2b[...] + y2p[...]) + b2[...]
    h2_o[...] = h2
    y3p_o[...] = dinv2[...] * _mm(h2, wcat[...])


def _log_softmax16(z):
    m = jnp.max(z, axis=1, keepdims=True)
    e = jnp.exp(z - m)
    return z - m - jnp.log(jnp.sum(e, axis=1, keepdims=True))


def _post_body(s3a, s3b, y3p, dinv2, bcat, attr_o, att_o):
    z = dinv2[...] * (s3a[...] + s3b[...] + y3p[...]) + bcat[...]
    attr_o[...] = _log_softmax16(z[:, :16])
    att_o[...] = _log_softmax16(z[:, 16:])


_BR = 1000
_NB = N_NODES // _BR


def _row_spec(cols):
    return pl.BlockSpec((_BR, cols), lambda i: (i, 0))


def _full_spec(r, c):
    return pl.BlockSpec((r, c), lambda i: (0, 0))


def _pad_reshape(a, per_tile_chunks, fill):
    total = NW * per_tile_chunks * ECH
    a = jnp.pad(a, (0, total - a.shape[0]), constant_values=fill)
    return a.reshape(NW, per_tile_chunks, ECH)


def kernel(x, train_pos_edge_index, edge_index, edge_weight, pos_edge_index,
           neg_edge_index, W1, b1, W2, b2, W_attr, b_attr, W_att, b_att):
    tpei = train_pos_edge_index.astype(I32)
    ei = edge_index.astype(I32)
    src1 = _pad_reshape(tpei[0], CH_E, 0)
    dst1 = _pad_reshape(tpei[1], CH_E, N_NODES)
    src2 = _pad_reshape(ei[0], CH_E, 0)
    dst2 = _pad_reshape(ei[1], CH_E, N_NODES)
    wpad = _pad_reshape(edge_weight.astype(F32), CH_E, 0.0)
    tot = jnp.concatenate([pos_edge_index, neg_edge_index], axis=-1).astype(I32)
    si = _pad_reshape(tot[0], CH_L, 0)
    di = _pad_reshape(tot[1], CH_L, 0)

    deg1p, deg2p = _deg_call(dst1, dst2, wpad)
    dinv1r, dinv2r = pl.pallas_call(
        _dinv_body,
        out_shape=(_sds((1, N_PAD), F32), _sds((1, N_PAD), F32)),
    )(deg1p, deg2p)
    dinv1 = dinv1r.reshape(N_PAD, 1)[:N_NODES]
    dinv2 = dinv2r.reshape(N_PAD, 1)[:N_NODES]

    xw1p = pl.pallas_call(
        _lin1_body, out_shape=_sds((N_NODES, 128), F32),
        grid=(_NB,),
        in_specs=[_row_spec(128), _full_spec(128, 128), _row_spec(1)],
        out_specs=_row_spec(128),
    )(x, W1, dinv1)
    s1 = _prop_call(128, src1, dst1, xw1p)
    y2p = pl.pallas_call(
        _lin2_body, out_shape=_sds((N_NODES, 64), F32),
        grid=(_NB,),
        in_specs=[_row_spec(128), _row_spec(128), _row_spec(128),
                  _row_spec(1), _full_spec(1, 128), _full_spec(128, 64)],
        out_specs=_row_spec(64),
    )(s1[0, :N_NODES], s1[1, :N_NODES], xw1p, dinv1, b1.reshape(1, -1), W2)

    s2 = _prop_call(64, src1, dst1, y2p)
    wcat = jnp.concatenate([W_attr, W_att], axis=1)
    bcat = jnp.concatenate([b_attr, b_att]).reshape(1, -1)
    h2, y3p = pl.pallas_call(
        _lin3_body,
        out_shape=(_sds((N_NODES, 64), F32), _sds((N_NODES, 32), F32)),
        grid=(_NB,),
        in_specs=[_row_spec(64), _row_spec(64), _row_spec(64), _row_spec(1),
                  _full_spec(1, 64), _full_spec(64, 32), _row_spec(1)],
        out_specs=(_row_spec(64), _row_spec(32)),
    )(s2[0, :N_NODES], s2[1, :N_NODES], y2p, dinv1, b2.reshape(1, -1), wcat,
      dinv2)

    s3, res3 = _prop3dot_call(src2, dst2, wpad, y3p, si, di, h2)
    attr, att = pl.pallas_call(
        _post_body,
        out_shape=(_sds((N_NODES, 16), F32), _sds((N_NODES, 16), F32)),
        grid=(_NB,),
        in_specs=[_row_spec(32), _row_spec(32), _row_spec(32), _row_spec(1),
                  _full_spec(1, 32)],
        out_specs=(_row_spec(16), _row_spec(16)),
    )(s3[0, :N_NODES], s3[1, :N_NODES], y3p, dinv2, bcat)

    res = res3.reshape(-1)[:EL_TOT]
    return (res, attr, att)


# speedup vs baseline: 20.6586x; 1.2899x over previous
import functools

import jax
import jax.numpy as jnp
from jax import lax
from jax.experimental import pallas as pl
from jax.experimental.pallas import tpu as pltpu
from jax.experimental.pallas import tpu_sc as plsc

N_NODES = 10000
N_PAD = 10112
NCORE = 2
NSUB = 16
NW = NCORE * NSUB
STRIPE = N_PAD // NSUB
ECH = 128
E_TOT = 320000
CH_E = 79
EPAD_E = NW * CH_E * ECH
EL_TOT = 200000
CH_L = 49
EPAD_L = NW * CH_L * ECH

F32 = jnp.float32
I32 = jnp.int32

_MESH = plsc.VectorSubcoreMesh(core_axis_name="c", subcore_axis_name="s")
_SC_PARAMS = pltpu.CompilerParams(needs_layout_passes=False,
                                  use_tc_tiling_on_sc=False)


def _sds(shape, dtype):
    return jax.ShapeDtypeStruct(shape, dtype)


def _zero_buf(buf, nrow, ncol16):
    z = jnp.zeros((16,), F32)

    def body(r, c):
        for k in range(ncol16):
            buf[r, pl.ds(k * 16, 16)] = z
        return c

    lax.fori_loop(0, nrow, body, 0)


def _zero_stripe(rows, acc, sid, zr):
    base = pl.multiple_of(sid * STRIPE, 8)
    n_full = STRIPE // zr
    for k in range(n_full):
        pltpu.sync_copy(rows, acc.at[pl.ds(base + k * zr, zr)])
    rem = STRIPE - n_full * zr
    if rem:
        pltpu.sync_copy(rows.at[pl.ds(0, rem)],
                        acc.at[pl.ds(base + n_full * zr, rem)])


def _dump_stripe(acc, part, cid, sid):
    base = pl.multiple_of(sid * STRIPE, 8)
    for k in range(4):
        pltpu.sync_copy(acc.at[pl.ds(base + k * 128, 128)],
                        part.at[cid].at[pl.ds(base + k * 128, 128)])
    rem = STRIPE - 512
    pltpu.sync_copy(acc.at[pl.ds(base + 512, rem)],
                    part.at[cid].at[pl.ds(base + 512, rem)])


def _deg_body(d1_h, d2_h, w_h, deg1_h, deg2_h, buf1, buf2, bufw, ldeg1, ldeg2):
    cid = lax.axis_index("c")
    sid = lax.axis_index("s")
    wid = cid * NSUB + sid
    pltpu.sync_copy(d1_h.at[wid], buf1)
    pltpu.sync_copy(d2_h.at[wid], buf2)
    pltpu.sync_copy(w_h.at[wid], bufw)
    z = jnp.zeros((16,), F32)

    def zb(i, c):
        ldeg1[pl.ds(i * 16, 16)] = z
        ldeg2[pl.ds(i * 16, 16)] = z
        return c

    lax.fori_loop(0, N_PAD // 16, zb, 0)
    ones = jnp.ones((16,), F32)

    def eb(j, c):
        for k in range(8):
            i1 = buf1[j, pl.ds(k * 16, 16)]
            plsc.addupdate_scatter(ldeg1, [i1], ones)
            i2 = buf2[j, pl.ds(k * 16, 16)]
            wv = bufw[j, pl.ds(k * 16, 16)]
            plsc.addupdate_scatter(ldeg2, [i2], wv)
        return c

    lax.fori_loop(0, CH_E, eb, 0)
    pltpu.sync_copy(ldeg1, deg1_h.at[wid])
    pltpu.sync_copy(ldeg2, deg2_h.at[wid])


def _deg_call(dst1, dst2, wpad):
    return pl.kernel(
        _deg_body,
        out_type=(_sds((NW, N_PAD), F32), _sds((NW, N_PAD), F32)),
        mesh=_MESH,
        compiler_params=_SC_PARAMS,
        scratch_types=[
            pltpu.VMEM((CH_E, ECH), I32),
            pltpu.VMEM((CH_E, ECH), I32),
            pltpu.VMEM((CH_E, ECH), F32),
            pltpu.VMEM((N_PAD,), F32),
            pltpu.VMEM((N_PAD,), F32),
        ],
    )(dst1, dst2, wpad)


def _make_prop_body(F, ech, ch):
    assert ch % 2 == 1
    ncol16 = F // 16

    def body(src_h, dst_h, y_h, part_h, sidx, didx, rows_a, rows_b, acc,
             sem_a, sem_b):
        cid = lax.axis_index("c")
        sid = lax.axis_index("s")
        wid = cid * NSUB + sid
        pltpu.sync_copy(src_h.at[wid], sidx)
        pltpu.sync_copy(dst_h.at[wid], didx)
        _zero_buf(rows_a, ech, ncol16)
        _zero_stripe(rows_a, acc, sid, ech)
        plsc.subcore_barrier()

        def gstart(j, buf, sem):
            pltpu.async_copy(y_h.at[sidx.at[j]], buf, sem)

        def gwait(j, buf, sem):
            pltpu.make_async_copy(y_h.at[sidx.at[j]], buf, sem).wait()

        def scat(j, buf):
            pltpu.sync_copy(buf, acc.at[didx.at[j]], add=True)

        gstart(0, rows_a, sem_a)

        def pair(i, c):
            j = 2 * i
            gwait(j, rows_a, sem_a)
            gstart(j + 1, rows_b, sem_b)
            scat(j, rows_a)
            gwait(j + 1, rows_b, sem_b)
            gstart(j + 2, rows_a, sem_a)
            scat(j + 1, rows_b)
            return c

        lax.fori_loop(0, (ch - 1) // 2, pair, 0)
        gwait(ch - 1, rows_a, sem_a)
        scat(ch - 1, rows_a)
        plsc.subcore_barrier()
        _dump_stripe(acc, part_h, cid, sid)

    return body


def _prop_call(F, ech, ch, src, dst, y):
    return pl.kernel(
        _make_prop_body(F, ech, ch),
        out_type=_sds((NCORE, N_PAD, F), F32),
        mesh=_MESH,
        compiler_params=_SC_PARAMS,
        scratch_types=[
            pltpu.VMEM((ch, ech), I32),
            pltpu.VMEM((ch, ech), I32),
            pltpu.VMEM((ech, F), F32),
            pltpu.VMEM((ech, F), F32),
            pltpu.VMEM_SHARED((N_PAD, F), F32),
            pltpu.SemaphoreType.DMA,
            pltpu.SemaphoreType.DMA,
        ],
    )(src, dst, y)


def _prop3dot_body(src_h, dst_h, w_h, y_h, si_h, di_h, h2_h, part_h, res_h,
                   sidx, didx, wbuf, rows_a, rows_b, si, di,
                   rs_a, rd_a, rs_b, rd_b, rbuf, acc, sem_a, sem_b):
    cid = lax.axis_index("c")
    sid = lax.axis_index("s")
    wid = cid * NSUB + sid
    pltpu.sync_copy(src_h.at[wid], sidx)
    pltpu.sync_copy(dst_h.at[wid], didx)
    pltpu.sync_copy(w_h.at[wid], wbuf)
    pltpu.sync_copy(si_h.at[wid], si)
    pltpu.sync_copy(di_h.at[wid], di)
    _zero_buf(rows_a, ECH, 2)
    _zero_stripe(rows_a, acc, sid, ECH)
    plsc.subcore_barrier()

    def gstart(j, buf, sem):
        pltpu.async_copy(y_h.at[sidx.at[j]], buf, sem)

    def gwait(j, buf, sem):
        pltpu.make_async_copy(y_h.at[sidx.at[j]], buf, sem).wait()

    def scale_scatter(j, rows):
        def sgroup(g, c2):
            wv = wbuf[j, pl.ds(g * 16, 16)]
            for l in range(16):
                w = wv[l]
                r = g * 16 + l
                for k in range(2):
                    rows[r, pl.ds(k * 16, 16)] = rows[r, pl.ds(k * 16, 16)] * w
            return c2

        lax.fori_loop(0, ECH // 16, sgroup, 0)
        pltpu.sync_copy(rows, acc.at[didx.at[j]], add=True)

    gstart(0, rows_a, sem_a)

    def pair(i, c):
        j = 2 * i
        gwait(j, rows_a, sem_a)
        gstart(j + 1, rows_b, sem_b)
        scale_scatter(j, rows_a)
        gwait(j + 1, rows_b, sem_b)
        gstart(j + 2, rows_a, sem_a)
        scale_scatter(j + 1, rows_b)
        return c

    lax.fori_loop(0, (CH_E - 1) // 2, pair, 0)
    gwait(CH_E - 1, rows_a, sem_a)
    scale_scatter(CH_E - 1, rows_a)

    lane = lax.iota(I32, 16)

    def dstart(j, bs, bd, sem):
        pltpu.async_copy(h2_h.at[si.at[j]], bs, sem)
        pltpu.async_copy(h2_h.at[di.at[j]], bd, sem)

    def dwait(j, bs, bd, sem):
        pltpu.make_async_copy(h2_h.at[si.at[j]], bs, sem).wait()
        pltpu.make_async_copy(h2_h.at[di.at[j]], bd, sem).wait()

    def dcompute(j, bs, bd):
        def group(g, c2):
            res = jnp.zeros((16,), F32)
            for l in range(16):
                e = g * 16 + l
                av = bs[e, pl.ds(0, 16)] * bd[e, pl.ds(0, 16)]
                for k in range(1, 4):
                    av = av + bs[e, pl.ds(k * 16, 16)] * bd[e, pl.ds(k * 16, 16)]
                s = jnp.sum(av)
                res = jnp.where(lane == l, s, res)
            rbuf[j, pl.ds(g * 16, 16)] = res
            return c2

        lax.fori_loop(0, ECH // 16, group, 0)

    dstart(0, rs_a, rd_a, sem_a)

    def dpair(i, c):
        j = 2 * i
        dwait(j, rs_a, rd_a, sem_a)
        dstart(j + 1, rs_b, rd_b, sem_b)
        dcompute(j, rs_a, rd_a)
        dwait(j + 1, rs_b, rd_b, sem_b)
        dstart(j + 2, rs_a, rd_a, sem_a)
        dcompute(j + 1, rs_b, rd_b)
        return c

    lax.fori_loop(0, (CH_L - 1) // 2, dpair, 0)
    dwait(CH_L - 1, rs_a, rd_a, sem_a)
    dcompute(CH_L - 1, rs_a, rd_a)
    pltpu.sync_copy(rbuf, res_h.at[wid])

    plsc.subcore_barrier()
    _dump_stripe(acc, part_h, cid, sid)


def _prop3dot_call(src, dst, wpad, y, si, di, h2):
    return pl.kernel(
        _prop3dot_body,
        out_type=(_sds((NCORE, N_PAD, 32), F32), _sds((NW, CH_L, ECH), F32)),
        mesh=_MESH,
        compiler_params=_SC_PARAMS,
        scratch_types=[
            pltpu.VMEM((CH_E, ECH), I32),
            pltpu.VMEM((CH_E, ECH), I32),
            pltpu.VMEM((CH_E, ECH), F32),
            pltpu.VMEM((ECH, 32), F32),
            pltpu.VMEM((ECH, 32), F32),
            pltpu.VMEM((CH_L, ECH), I32),
            pltpu.VMEM((CH_L, ECH), I32),
            pltpu.VMEM((ECH, 64), F32),
            pltpu.VMEM((ECH, 64), F32),
            pltpu.VMEM((ECH, 64), F32),
            pltpu.VMEM((ECH, 64), F32),
            pltpu.VMEM((CH_L, ECH), F32),
            pltpu.VMEM_SHARED((N_PAD, 32), F32),
            pltpu.SemaphoreType.DMA,
            pltpu.SemaphoreType.DMA,
        ],
    )(src, dst, wpad, y, si, di, h2)


_HIGH = lax.Precision.HIGHEST


def _mm(a, b):
    return jnp.dot(a, b, precision=_HIGH, preferred_element_type=F32)


def _dinv_body(d1_ref, d2_ref, o1_ref, o2_ref):
    o1_ref[...] = lax.rsqrt(jnp.sum(d1_ref[...], axis=0, keepdims=True) + 1.0)
    o2_ref[...] = lax.rsqrt(jnp.sum(d2_ref[...], axis=0, keepdims=True) + 1.0)


def _lin1_body(x_ref, w1_ref, dinv_ref, o_ref):
    o_ref[...] = dinv_ref[...] * _mm(x_ref[...], w1_ref[...])


def _lin2_body(s1a, s1b, xw1p, dinv, b1, w2, o_ref):
    h1 = jax.nn.relu(dinv[...] * (s1a[...] + s1b[...] + xw1p[...]) + b1[...])
    o_ref[...] = dinv[...] * _mm(h1, w2[...])


def _lin3_body(s2a, s2b, y2p, dinv1, b2, wcat, dinv2, h2_o, y3p_o):
    h2 = dinv1[...] * (s2a[...] + s2b[...] + y2p[...]) + b2[...]
    h2_o[...] = h2
    y3p_o[...] = dinv2[...] * _mm(h2, wcat[...])


def _log_softmax16(z):
    m = jnp.max(z, axis=1, keepdims=True)
    e = jnp.exp(z - m)
    return z - m - jnp.log(jnp.sum(e, axis=1, keepdims=True))


def _post_body(s3a, s3b, y3p, dinv2, bcat, attr_o, att_o):
    z = dinv2[...] * (s3a[...] + s3b[...] + y3p[...]) + bcat[...]
    attr_o[...] = _log_softmax16(z[:, :16])
    att_o[...] = _log_softmax16(z[:, 16:])


_BR = 1000
_NB = N_NODES // _BR


def _row_spec(cols):
    return pl.BlockSpec((_BR, cols), lambda i: (i, 0))


def _full_spec(r, c):
    return pl.BlockSpec((r, c), lambda i: (0, 0))


def _pad_reshape(a, per_tile_chunks, ech, fill):
    total = NW * per_tile_chunks * ech
    a = jnp.pad(a, (0, total - a.shape[0]), constant_values=fill)
    return a.reshape(NW, per_tile_chunks, ech)


def kernel(x, train_pos_edge_index, edge_index, edge_weight, pos_edge_index,
           neg_edge_index, W1, b1, W2, b2, W_attr, b_attr, W_att, b_att):
    tpei = train_pos_edge_index.astype(I32)
    ei = edge_index.astype(I32)
    src1 = _pad_reshape(tpei[0], CH_E, ECH, 0)
    dst1 = _pad_reshape(tpei[1], CH_E, ECH, N_NODES)
    src1n = _pad_reshape(tpei[0], 157, 64, 0)
    dst1n = _pad_reshape(tpei[1], 157, 64, N_NODES)
    src2 = _pad_reshape(ei[0], CH_E, ECH, 0)
    dst2 = _pad_reshape(ei[1], CH_E, ECH, N_NODES)
    wpad = _pad_reshape(edge_weight.astype(F32), CH_E, ECH, 0.0)
    tot = jnp.concatenate([pos_edge_index, neg_edge_index], axis=-1).astype(I32)
    si = _pad_reshape(tot[0], CH_L, ECH, 0)
    di = _pad_reshape(tot[1], CH_L, ECH, 0)

    deg1p, deg2p = _deg_call(dst1, dst2, wpad)
    dinv1r, dinv2r = pl.pallas_call(
        _dinv_body,
        out_shape=(_sds((1, N_PAD), F32), _sds((1, N_PAD), F32)),
    )(deg1p, deg2p)
    dinv1 = dinv1r.reshape(N_PAD, 1)[:N_NODES]
    dinv2 = dinv2r.reshape(N_PAD, 1)[:N_NODES]

    xw1p = pl.pallas_call(
        _lin1_body, out_shape=_sds((N_NODES, 128), F32),
        grid=(_NB,),
        in_specs=[_row_spec(128), _full_spec(128, 128), _row_spec(1)],
        out_specs=_row_spec(128),
    )(x, W1, dinv1)
    s1 = _prop_call(128, 64, 157, src1n, dst1n, xw1p)
    y2p = pl.pallas_call(
        _lin2_body, out_shape=_sds((N_NODES, 64), F32),
        grid=(_NB,),
        in_specs=[_row_spec(128), _row_spec(128), _row_spec(128),
                  _row_spec(1), _full_spec(1, 128), _full_spec(128, 64)],
        out_specs=_row_spec(64),
    )(s1[0, :N_NODES], s1[1, :N_NODES], xw1p, dinv1, b1.reshape(1, -1), W2)

    s2 = _prop_call(64, ECH, CH_E, src1, dst1, y2p)
    wcat = jnp.concatenate([W_attr, W_att], axis=1)
    bcat = jnp.concatenate([b_attr, b_att]).reshape(1, -1)
    h2, y3p = pl.pallas_call(
        _lin3_body,
        out_shape=(_sds((N_NODES, 64), F32), _sds((N_NODES, 32), F32)),
        grid=(_NB,),
        in_specs=[_row_spec(64), _row_spec(64), _row_spec(64), _row_spec(1),
                  _full_spec(1, 64), _full_spec(64, 32), _row_spec(1)],
        out_specs=(_row_spec(64), _row_spec(32)),
    )(s2[0, :N_NODES], s2[1, :N_NODES], y2p, dinv1, b2.reshape(1, -1), wcat,
      dinv2)

    s3, res3 = _prop3dot_call(src2, dst2, wpad, y3p, si, di, h2)
    attr, att = pl.pallas_call(
        _post_body,
        out_shape=(_sds((N_NODES, 16), F32), _sds((N_NODES, 16), F32)),
        grid=(_NB,),
        in_specs=[_row_spec(32), _row_spec(32), _row_spec(32), _row_spec(1),
                  _full_spec(1, 32)],
        out_specs=(_row_spec(16), _row_spec(16)),
    )(s3[0, :N_NODES], s3[1, :N_NODES], y3p, dinv2, bcat)

    res = res3.reshape(-1)[:EL_TOT]
    return (res, attr, att)
